# baseline (speedup 1.0000x reference)
import functools

import jax
import jax.numpy as jnp
from jax.experimental import pallas as pl
from jax.experimental.pallas import tpu as pltpu


def _se_body(x_ref, w1t_ref, w2t_ref, y_ref, *, inv_hw):
    xb = x_ref[...]
    mean = jnp.sum(xb, axis=-1) * inv_hw
    h = jnp.maximum(
        jnp.dot(mean, w1t_ref[...], preferred_element_type=jnp.float32), 0.0)
    gate = jax.nn.sigmoid(
        jnp.dot(h, w2t_ref[...], preferred_element_type=jnp.float32))
    y_ref[...] = xb * gate[:, :, None]


def _pick_batch_tile(B, slab_bytes, budget_bytes):
    for cand in range(B, 0, -1):
        if B % cand:
            continue
        steps = B // cand
        if steps >= 2 and steps % 2 == 0 and cand * slab_bytes <= budget_bytes:
            return cand
    return 1


def kernel(x, w1, w2):
    B, C, H, W = x.shape
    HW = H * W
    hidden = w1.shape[0]
    itemsize = jnp.dtype(x.dtype).itemsize
    inv_hw = 1.0 / float(HW)

    x_flat = x.reshape(B, C, HW)
    w1t = jnp.transpose(w1)
    w2t = jnp.transpose(w2)

    slab_bytes = C * HW * itemsize
    bt = _pick_batch_tile(B, slab_bytes, budget_bytes=7 * 1024 * 1024)
    n_steps = pl.cdiv(B, bt)

    cost = pl.CostEstimate(
        flops=2 * B * C * hidden * 2 + 2 * B * C * HW,
        transcendentals=B * C,
        bytes_accessed=2 * B * C * HW * itemsize,
    )

    y_flat = pl.pallas_call(
        functools.partial(_se_body, inv_hw=inv_hw),
        out_shape=jax.ShapeDtypeStruct((B, C, HW), x.dtype),
        grid=(n_steps,),
        in_specs=[
            pl.BlockSpec((bt, C, HW), lambda b: (b, 0, 0)),
            pl.BlockSpec((C, hidden), lambda b: (0, 0)),
            pl.BlockSpec((hidden, C), lambda b: (0, 0)),
        ],
        out_specs=pl.BlockSpec((bt, C, HW), lambda b: (b, 0, 0)),
        compiler_params=pltpu.CompilerParams(
            dimension_semantics=("parallel",),
            vmem_limit_bytes=56 * 1024 * 1024,
        ),
        cost_estimate=cost,
    )(x_flat, w1t, w2t)

    return y_flat.reshape(B, C, H, W)


# speedup vs baseline: 1.0728x; 1.0728x over previous
import functools

import jax
import jax.numpy as jnp
from jax.experimental import pallas as pl
from jax.experimental.pallas import tpu as pltpu


def _se_body(x_ref, w1t_ref, w2t_ref, y_ref, *, inv_hw):
    xb = x_ref[...]
    pooled = jnp.sum(xb, axis=(1, 2)) * inv_hw
    h = jnp.maximum(
        jnp.dot(pooled, w1t_ref[...], preferred_element_type=jnp.float32), 0.0)
    gate = jax.nn.sigmoid(
        jnp.dot(h, w2t_ref[...], preferred_element_type=jnp.float32))
    y_ref[...] = xb * gate[:, None, None, :]


def _pick_batch_tile(B, slab_bytes, budget_bytes):
    for cand in range(B, 0, -1):
        if B % cand:
            continue
        steps = B // cand
        if steps >= 2 and steps % 2 == 0 and cand * slab_bytes <= budget_bytes:
            return cand
    return 1


def kernel(x, w1, w2):
    B, C, H, W = x.shape
    HW = H * W
    hidden = w1.shape[0]
    itemsize = jnp.dtype(x.dtype).itemsize
    inv_hw = 1.0 / float(HW)

    x_t = jnp.transpose(x, (0, 2, 3, 1))
    w1t = jnp.transpose(w1)
    w2t = jnp.transpose(w2)

    w_pad = -(-W // 8) * 8
    slab_bytes = H * w_pad * C * itemsize
    bt = _pick_batch_tile(B, slab_bytes, budget_bytes=8 * 1024 * 1024)
    n_steps = pl.cdiv(B, bt)

    cost = pl.CostEstimate(
        flops=2 * B * C * hidden * 2 + 2 * B * C * HW,
        transcendentals=B * C,
        bytes_accessed=2 * B * C * HW * itemsize,
    )

    y_t = pl.pallas_call(
        functools.partial(_se_body, inv_hw=inv_hw),
        out_shape=jax.ShapeDtypeStruct((B, H, W, C), x.dtype),
        grid=(n_steps,),
        in_specs=[
            pl.BlockSpec((bt, H, W, C), lambda b: (b, 0, 0, 0)),
            pl.BlockSpec((C, hidden), lambda b: (0, 0)),
            pl.BlockSpec((hidden, C), lambda b: (0, 0)),
        ],
        out_specs=pl.BlockSpec((bt, H, W, C), lambda b: (b, 0, 0, 0)),
        compiler_params=pltpu.CompilerParams(
            dimension_semantics=("parallel",),
            vmem_limit_bytes=56 * 1024 * 1024,
        ),
        cost_estimate=cost,
    )(x_t, w1t, w2t)

    return jnp.transpose(y_t, (0, 3, 1, 2))


# speedup vs baseline: 4.4986x; 4.1935x over previous
import functools

import jax
import jax.numpy as jnp
from jax.experimental import pallas as pl
from jax.experimental.pallas import tpu as pltpu


def _se_body(x_ref, w1t_ref, w2t_ref, y_ref, *, inv_hw):
    xb = x_ref[...]
    pooled = jnp.sum(xb, axis=(0, 1)) * inv_hw
    h = jnp.maximum(
        jnp.dot(pooled, w1t_ref[...], preferred_element_type=jnp.float32), 0.0)
    gate = jax.nn.sigmoid(
        jnp.dot(h, w2t_ref[...], preferred_element_type=jnp.float32))
    y_ref[...] = xb * gate[None, None, :, :]


def _pick_batch_tile(B, slab_bytes, budget_bytes):
    for cand in range(B, 0, -1):
        if B % cand or cand % 8:
            continue
        steps = B // cand
        if steps >= 2 and steps % 2 == 0 and cand * slab_bytes <= budget_bytes:
            return cand
    return min(B, 8)


def kernel(x, w1, w2):
    B, C, H, W = x.shape
    HW = H * W
    hidden = w1.shape[0]
    itemsize = jnp.dtype(x.dtype).itemsize
    inv_hw = 1.0 / float(HW)

    x_p = jnp.transpose(x, (2, 3, 0, 1))
    w1t = jnp.transpose(w1)
    w2t = jnp.transpose(w2)

    slab_bytes = H * W * C * itemsize
    bt = _pick_batch_tile(B, slab_bytes, budget_bytes=13 * 1024 * 1024)
    n_steps = pl.cdiv(B, bt)

    cost = pl.CostEstimate(
        flops=2 * B * C * hidden * 2 + 2 * B * C * HW,
        transcendentals=B * C,
        bytes_accessed=2 * B * C * HW * itemsize,
    )

    y_p = pl.pallas_call(
        functools.partial(_se_body, inv_hw=inv_hw),
        out_shape=jax.ShapeDtypeStruct((H, W, B, C), x.dtype),
        grid=(n_steps,),
        in_specs=[
            pl.BlockSpec((H, W, bt, C), lambda b: (0, 0, b, 0)),
            pl.BlockSpec((C, hidden), lambda b: (0, 0)),
            pl.BlockSpec((hidden, C), lambda b: (0, 0)),
        ],
        out_specs=pl.BlockSpec((H, W, bt, C), lambda b: (0, 0, b, 0)),
        compiler_params=pltpu.CompilerParams(
            dimension_semantics=("parallel",),
            vmem_limit_bytes=56 * 1024 * 1024,
        ),
        cost_estimate=cost,
    )(x_p, w1t, w2t)

    return jnp.transpose(y_p, (2, 3, 0, 1))
